# baseline (speedup 1.0000x reference)
import jax
import jax.numpy as jnp
from jax.experimental import pallas as pl
from jax.experimental.pallas import tpu as pltpu

_N = 32768
_M = 1024
_L = 512
_S = 16
_BN = 2048
_NB = _N // _BN
_CN = 2048


def _fused_body(x_ref, seg_ref, wcat_ref, bcat_ref, ww_ref, bw_ref,
                out_ref, z_acc, s_acc):
    i = pl.program_id(0)

    @pl.when(i == 0)
    def _init():
        z_acc[...] = jnp.zeros_like(z_acc)
        s_acc[...] = jnp.zeros_like(s_acc)

    neg_inf = jnp.float32(-jnp.inf)
    nc = _BN // _CN
    seg_row = seg_ref[0]
    xs = [x_ref[pl.ds(c * _CN, _CN), :].astype(jnp.bfloat16) for c in range(nc)]
    hs = [jax.lax.dot_general(
        xs[c], wcat_ref[...], (((1,), (0,)), ((), ())),
        preferred_element_type=jnp.float32) + bcat_ref[...] for c in range(nc)]
    gs = [jnp.tanh(h[:, :_L]) * (1.0 + jnp.tanh(h[:, _L:])) for h in hs]
    a_rows = [jax.lax.dot_general(
        ww_ref[...], g, (((1,), (1,)), ((), ())),
        preferred_element_type=jnp.float32) + bw_ref[...] for g in gs]
    seg_iota = jax.lax.broadcasted_iota(jnp.int32, (_S, _CN), 0)
    masks = [seg_iota == seg_row[:, c * _CN:(c + 1) * _CN] for c in range(nc)]

    s_blk = jnp.float32(0.0)
    z_blk = jnp.zeros((_S, _M), jnp.float32)
    for c in range(nc):
        e = jnp.exp(a_rows[c])
        p = jnp.where(masks[c], e, 0.0)
        s_blk = s_blk + jnp.sum(p, axis=1, keepdims=True)
        z_blk = z_blk + jax.lax.dot_general(
            p.astype(jnp.bfloat16), xs[c], (((1,), (0,)), ((), ())),
            preferred_element_type=jnp.float32)

    s_acc[...] = s_acc[...] + s_blk
    z_acc[...] = z_acc[...] + z_blk

    @pl.when(i == _NB - 1)
    def _finish():
        s = s_acc[...]
        denom = jnp.where(s == 0.0, 1.0, s)
        out_ref[...] = z_acc[...] / denom


def kernel(x, batch, Wv, bv, Wu, bu, Ww, bw):
    wcat = jnp.concatenate([Wv.T, 0.5 * Wu.T], axis=1).astype(jnp.bfloat16)

    bcat = jnp.concatenate([bv, 0.5 * bu]).reshape(1, 2 * _L)
    seg3 = batch.astype(jnp.int32).reshape(_NB, 1, _BN)
    ww = (0.5 * Ww).reshape(1, _L)
    bwr = bw.reshape(1, 1)

    grid = (_NB,)
    out = pl.pallas_call(
        _fused_body,
        grid=grid,
        in_specs=[
            pl.BlockSpec((_BN, _M), lambda i: (i, 0)),
            pl.BlockSpec((1, 1, _BN), lambda i: (i, 0, 0)),
            pl.BlockSpec((_M, 2 * _L), lambda i: (0, 0)),
            pl.BlockSpec((1, 2 * _L), lambda i: (0, 0)),
            pl.BlockSpec((1, _L), lambda i: (0, 0)),
            pl.BlockSpec((1, 1), lambda i: (0, 0)),
        ],
        out_specs=pl.BlockSpec((_S, _M), lambda i: (0, 0)),
        out_shape=jax.ShapeDtypeStruct((_S, _M), jnp.float32),
        scratch_shapes=[
            pltpu.VMEM((_S, _M), jnp.float32),
            pltpu.VMEM((_S, 1), jnp.float32),
        ],
        compiler_params=pltpu.CompilerParams(
            dimension_semantics=("arbitrary",),
        ),
    )(x, seg3, wcat, bcat, ww, bwr)
    return out


# speedup vs baseline: 1.0540x; 1.0540x over previous
import jax
import jax.numpy as jnp
from jax.experimental import pallas as pl
from jax.experimental.pallas import tpu as pltpu

_N = 32768
_M = 1024
_L = 512
_S = 16
_BN = 2048
_NB = _N // _BN
_CN = 1024


def _fused_body(x_ref, seg_ref, wcat_ref, bcat_ref, ww_ref, bw_ref,
                out_ref, z_acc, s_acc):
    i = pl.program_id(0)

    @pl.when(i == 0)
    def _init():
        z_acc[...] = jnp.zeros_like(z_acc)
        s_acc[...] = jnp.zeros_like(s_acc)

    neg_inf = jnp.float32(-jnp.inf)
    nc = _BN // _CN
    seg_row = seg_ref[0]
    xs = [x_ref[pl.ds(c * _CN, _CN), :].astype(jnp.bfloat16) for c in range(nc)]
    hs = [jax.lax.dot_general(
        xs[c], wcat_ref[...], (((1,), (0,)), ((), ())),
        preferred_element_type=jnp.float32) + bcat_ref[...] for c in range(nc)]
    gs = [jnp.tanh(h[:, :_L]) * (1.0 + jnp.tanh(h[:, _L:])) for h in hs]
    a_rows = [jnp.transpose(
        jnp.sum(g * ww_ref[...], axis=1, keepdims=True) + bw_ref[...])
        for g in gs]
    seg_iota = jax.lax.broadcasted_iota(jnp.int32, (_S, _CN), 0)
    masks = [seg_iota == seg_row[:, c * _CN:(c + 1) * _CN] for c in range(nc)]

    s_blk = jnp.float32(0.0)
    z_blk = jnp.zeros((_S, _M), jnp.float32)
    for c in range(nc):
        e = jnp.exp(a_rows[c])
        p = jnp.where(masks[c], e, 0.0)
        s_blk = s_blk + jnp.sum(p, axis=1, keepdims=True)
        z_blk = z_blk + jax.lax.dot_general(
            p.astype(jnp.bfloat16), xs[c], (((1,), (0,)), ((), ())),
            preferred_element_type=jnp.float32)

    s_acc[...] = s_acc[...] + s_blk
    z_acc[...] = z_acc[...] + z_blk

    @pl.when(i == _NB - 1)
    def _finish():
        s = s_acc[...]
        denom = jnp.where(s == 0.0, 1.0, s)
        out_ref[...] = z_acc[...] / denom


def kernel(x, batch, Wv, bv, Wu, bu, Ww, bw):
    wcat = jnp.concatenate([Wv.T, 0.5 * Wu.T], axis=1).astype(jnp.bfloat16)

    bcat = jnp.concatenate([bv, 0.5 * bu]).reshape(1, 2 * _L)
    seg3 = batch.astype(jnp.int32).reshape(_NB, 1, _BN)
    ww = (0.5 * Ww).reshape(1, _L)
    bwr = bw.reshape(1, 1)

    grid = (_NB,)
    out = pl.pallas_call(
        _fused_body,
        grid=grid,
        in_specs=[
            pl.BlockSpec((_BN, _M), lambda i: (i, 0)),
            pl.BlockSpec((1, 1, _BN), lambda i: (i, 0, 0)),
            pl.BlockSpec((_M, 2 * _L), lambda i: (0, 0)),
            pl.BlockSpec((1, 2 * _L), lambda i: (0, 0)),
            pl.BlockSpec((1, _L), lambda i: (0, 0)),
            pl.BlockSpec((1, 1), lambda i: (0, 0)),
        ],
        out_specs=pl.BlockSpec((_S, _M), lambda i: (0, 0)),
        out_shape=jax.ShapeDtypeStruct((_S, _M), jnp.float32),
        scratch_shapes=[
            pltpu.VMEM((_S, _M), jnp.float32),
            pltpu.VMEM((_S, 1), jnp.float32),
        ],
        compiler_params=pltpu.CompilerParams(
            dimension_semantics=("arbitrary",),
        ),
    )(x, seg3, wcat, bcat, ww, bwr)
    return out


# speedup vs baseline: 1.0586x; 1.0043x over previous
import jax
import jax.numpy as jnp
from jax.experimental import pallas as pl
from jax.experimental.pallas import tpu as pltpu

_N = 32768
_M = 1024
_L = 512
_S = 16
_BN = 2048
_NB = _N // _BN
_CN = 1024


def _fused_body(x_ref, seg_ref, wcat_ref, bcat_ref, ww_ref, bw_ref,
                out_ref, z_acc, s_acc):
    i = pl.program_id(0)

    @pl.when(i == 0)
    def _init():
        z_acc[...] = jnp.zeros_like(z_acc)
        s_acc[...] = jnp.zeros_like(s_acc)

    neg_inf = jnp.float32(-jnp.inf)
    nc = _BN // _CN
    seg_row = seg_ref[0]
    xb = x_ref[...].astype(jnp.bfloat16)
    xs = [xb[c * _CN:(c + 1) * _CN, :] for c in range(nc)]
    hs = [jax.lax.dot_general(
        xs[c], wcat_ref[...], (((1,), (0,)), ((), ())),
        preferred_element_type=jnp.float32) + bcat_ref[...] for c in range(nc)]
    gs = [jnp.tanh(h[:, :_L]) * (1.0 + jnp.tanh(h[:, _L:])) for h in hs]
    a_rows = [jnp.transpose(
        jnp.sum(g * ww_ref[...], axis=1, keepdims=True) + bw_ref[...])
        for g in gs]
    seg_iota = jax.lax.broadcasted_iota(jnp.int32, (_S, _CN), 0)
    masks = [seg_iota == seg_row[:, c * _CN:(c + 1) * _CN] for c in range(nc)]

    ps = [jnp.where(masks[c], jnp.exp(a_rows[c]), 0.0) for c in range(nc)]
    p_full = jnp.concatenate(ps, axis=1)
    s_blk = jnp.sum(p_full, axis=1, keepdims=True)
    z_blk = jax.lax.dot_general(
        p_full.astype(jnp.bfloat16), xb, (((1,), (0,)), ((), ())),
        preferred_element_type=jnp.float32)

    s_acc[...] = s_acc[...] + s_blk
    z_acc[...] = z_acc[...] + z_blk

    @pl.when(i == _NB - 1)
    def _finish():
        s = s_acc[...]
        denom = jnp.where(s == 0.0, 1.0, s)
        out_ref[...] = z_acc[...] / denom


def kernel(x, batch, Wv, bv, Wu, bu, Ww, bw):
    wcat = jnp.concatenate([Wv.T, 0.5 * Wu.T], axis=1).astype(jnp.bfloat16)

    bcat = jnp.concatenate([bv, 0.5 * bu]).reshape(1, 2 * _L)
    seg3 = batch.astype(jnp.int32).reshape(_NB, 1, _BN)
    ww = (0.5 * Ww).reshape(1, _L)
    bwr = bw.reshape(1, 1)

    grid = (_NB,)
    out = pl.pallas_call(
        _fused_body,
        grid=grid,
        in_specs=[
            pl.BlockSpec((_BN, _M), lambda i: (i, 0)),
            pl.BlockSpec((1, 1, _BN), lambda i: (i, 0, 0)),
            pl.BlockSpec((_M, 2 * _L), lambda i: (0, 0)),
            pl.BlockSpec((1, 2 * _L), lambda i: (0, 0)),
            pl.BlockSpec((1, _L), lambda i: (0, 0)),
            pl.BlockSpec((1, 1), lambda i: (0, 0)),
        ],
        out_specs=pl.BlockSpec((_S, _M), lambda i: (0, 0)),
        out_shape=jax.ShapeDtypeStruct((_S, _M), jnp.float32),
        scratch_shapes=[
            pltpu.VMEM((_S, _M), jnp.float32),
            pltpu.VMEM((_S, 1), jnp.float32),
        ],
        compiler_params=pltpu.CompilerParams(
            dimension_semantics=("arbitrary",),
        ),
    )(x, seg3, wcat, bcat, ww, bwr)
    return out


# speedup vs baseline: 1.0589x; 1.0003x over previous
import jax
import jax.numpy as jnp
from jax.experimental import pallas as pl
from jax.experimental.pallas import tpu as pltpu

_N = 32768
_M = 1024
_L = 512
_S = 16
_BN = 2048
_NB = _N // _BN
_CN = 1024


def _fused_body(x_ref, seg_ref, wcat_ref, bcat_ref, ww_ref, bw_ref,
                out_ref, z_acc, s_acc):
    i = pl.program_id(0)

    @pl.when(i == 0)
    def _init():
        z_acc[...] = jnp.zeros_like(z_acc)
        s_acc[...] = jnp.zeros_like(s_acc)

    nc = _BN // _CN
    seg_row = seg_ref[0]
    xb = x_ref[...].astype(jnp.bfloat16)
    xs = [xb[c * _CN:(c + 1) * _CN, :] for c in range(nc)]
    hs = [jax.lax.dot_general(
        xs[c], wcat_ref[...], (((1,), (0,)), ((), ())),
        preferred_element_type=jnp.float32) + bcat_ref[...] for c in range(nc)]
    gs = [jnp.tanh(h[:, :_L]) * (1.0 + jnp.tanh(h[:, _L:])) for h in hs]
    a_rows = [jnp.transpose(
        jnp.sum(g * ww_ref[...], axis=1, keepdims=True) + bw_ref[...])
        for g in gs]
    seg_iota = jax.lax.broadcasted_iota(jnp.int32, (_S, _CN), 0)
    masks = [seg_iota == seg_row[:, c * _CN:(c + 1) * _CN] for c in range(nc)]

    ps = [jnp.where(masks[c], jnp.exp(a_rows[c]), 0.0) for c in range(nc)]
    p_full = jnp.concatenate(ps, axis=1)
    s_blk = jnp.sum(p_full, axis=1, keepdims=True)
    z_blk = jax.lax.dot_general(
        p_full.astype(jnp.bfloat16), xb, (((1,), (0,)), ((), ())),
        preferred_element_type=jnp.float32)

    s_acc[...] = s_acc[...] + s_blk
    z_acc[...] = z_acc[...] + z_blk

    @pl.when(i == _NB - 1)
    def _finish():
        s = s_acc[...]
        denom = jnp.where(s == 0.0, 1.0, s)
        out_ref[...] = z_acc[...] / denom


def kernel(x, batch, Wv, bv, Wu, bu, Ww, bw):
    wcat = jnp.concatenate([Wv.T, 0.5 * Wu.T], axis=1).astype(jnp.bfloat16)

    bcat = jnp.concatenate([bv, 0.5 * bu]).reshape(1, 2 * _L)
    seg3 = batch.astype(jnp.int32).reshape(_NB, 1, _BN)
    ww = (0.5 * Ww).reshape(1, _L)
    bwr = bw.reshape(1, 1)

    grid = (_NB,)
    out = pl.pallas_call(
        _fused_body,
        grid=grid,
        in_specs=[
            pl.BlockSpec((_BN, _M), lambda i: (i, 0)),
            pl.BlockSpec((1, 1, _BN), lambda i: (i, 0, 0)),
            pl.BlockSpec((_M, 2 * _L), lambda i: (0, 0)),
            pl.BlockSpec((1, 2 * _L), lambda i: (0, 0)),
            pl.BlockSpec((1, _L), lambda i: (0, 0)),
            pl.BlockSpec((1, 1), lambda i: (0, 0)),
        ],
        out_specs=pl.BlockSpec((_S, _M), lambda i: (0, 0)),
        out_shape=jax.ShapeDtypeStruct((_S, _M), jnp.float32),
        scratch_shapes=[
            pltpu.VMEM((_S, _M), jnp.float32),
            pltpu.VMEM((_S, 1), jnp.float32),
        ],
        compiler_params=pltpu.CompilerParams(
            dimension_semantics=("arbitrary",),
        ),
    )(x, seg3, wcat, bcat, ww, bwr)
    return out
